# Initial kernel scaffold; baseline (speedup 1.0000x reference)
import jax
import jax.numpy as jnp
from jax.experimental import pallas as pl


def kernel(xyz, points):
    raise NotImplementedError("write your pallas kernel here")


# speedup vs baseline: 1.6420x; 1.6420x over previous
import functools

import jax
import jax.numpy as jnp
from jax import lax
from jax.experimental import pallas as pl
from jax.experimental.pallas import tpu as pltpu

B, N, C = 8, 4096, 3
G = 512
K = 32


def _fps_body(xt_ref, idx_ref, cx_ref, cy_ref, cz_ref):
    x0 = xt_ref[0]
    x1 = xt_ref[1]
    x2 = xt_ref[2]
    iota = lax.broadcasted_iota(jnp.int32, (B, N), 1)
    rowoff = lax.broadcasted_iota(jnp.int32, (1, B), 1) * N

    def step(i, carry):
        dist, far = carry
        onehot = iota == far
        c0 = jnp.sum(jnp.where(onehot, x0, 0.0), axis=1, keepdims=True)
        c1 = jnp.sum(jnp.where(onehot, x1, 0.0), axis=1, keepdims=True)
        c2 = jnp.sum(jnp.where(onehot, x2, 0.0), axis=1, keepdims=True)
        idx_ref[pl.ds(i, 1), :] = far.reshape(1, B) + rowoff
        cx_ref[pl.ds(i, 1), :] = c0.reshape(1, B)
        cy_ref[pl.ds(i, 1), :] = c1.reshape(1, B)
        cz_ref[pl.ds(i, 1), :] = c2.reshape(1, B)
        d0 = (x0 - c0) ** 2
        d1 = (x1 - c1) ** 2
        d2 = (x2 - c2) ** 2
        d = (d0 + d1) + d2
        dist = jnp.minimum(dist, d)
        m = jnp.max(dist, axis=1, keepdims=True)
        cand = jnp.where(dist == m, iota, N)
        far = jnp.min(cand, axis=1, keepdims=True)
        return dist, far

    dist0 = jnp.full((B, N), 1e10, dtype=jnp.float32)
    far0 = jnp.zeros((B, 1), dtype=jnp.int32)
    lax.fori_loop(0, G, step, (dist0, far0))


def _fps(xt):
    out = pl.pallas_call(
        _fps_body,
        out_shape=(
            jax.ShapeDtypeStruct((G, B), jnp.int32),
            jax.ShapeDtypeStruct((G, B), jnp.float32),
            jax.ShapeDtypeStruct((G, B), jnp.float32),
            jax.ShapeDtypeStruct((G, B), jnp.float32),
        ),
    )(xt)
    return out


def kernel(xyz, points):
    xt = jnp.transpose(xyz, (2, 0, 1))
    fps_flat, cx, cy, cz = _fps(xt)
    fps_idx = (jnp.transpose(fps_flat) - jnp.arange(B)[:, None] * N).astype(jnp.int32)
    new_xyz = jnp.stack([cx.T, cy.T, cz.T], axis=-1)
    bidx = jnp.arange(B)[:, None]
    new_points = points[bidx, fps_idx, :]

    d = -2 * jnp.matmul(new_xyz, jnp.transpose(xyz, (0, 2, 1)))
    d = d + jnp.sum(new_xyz ** 2, -1).reshape(B, G, 1)
    d = d + jnp.sum(xyz ** 2, -1).reshape(B, 1, N)
    _, idx = jax.lax.top_k(-d, K)
    bidx3 = jnp.arange(B)[:, None, None]
    grouped_xyz = xyz[bidx3, idx, :]
    grouped_points = points[bidx3, idx, :]
    return (new_xyz, new_points, grouped_xyz, grouped_points)


# speedup vs baseline: 2.2073x; 1.3443x over previous
import functools

import jax
import jax.numpy as jnp
from jax import lax
from jax.experimental import pallas as pl
from jax.experimental.pallas import tpu as pltpu
from jax.experimental.pallas import tpu_sc as plsc

B, N, C = 8, 4096, 3
G = 512
K = 32

_SC_INFO = plsc.get_sparse_core_info()
NW = _SC_INFO.num_cores * _SC_INFO.num_subcores


def _fps_body(xt_ref, idx_ref, cx_ref, cy_ref, cz_ref):
    x0 = xt_ref[0]
    x1 = xt_ref[1]
    x2 = xt_ref[2]
    iota = lax.broadcasted_iota(jnp.int32, (B, N), 1)
    rowoff = lax.broadcasted_iota(jnp.int32, (1, B), 1) * N

    def step(i, carry):
        dist, far = carry
        onehot = iota == far
        c0 = jnp.sum(jnp.where(onehot, x0, 0.0), axis=1, keepdims=True)
        c1 = jnp.sum(jnp.where(onehot, x1, 0.0), axis=1, keepdims=True)
        c2 = jnp.sum(jnp.where(onehot, x2, 0.0), axis=1, keepdims=True)
        idx_ref[pl.ds(i, 1), :] = far.reshape(1, B) + rowoff
        cx_ref[pl.ds(i, 1), :] = c0.reshape(1, B)
        cy_ref[pl.ds(i, 1), :] = c1.reshape(1, B)
        cz_ref[pl.ds(i, 1), :] = c2.reshape(1, B)
        d0 = (x0 - c0) ** 2
        d1 = (x1 - c1) ** 2
        d2 = (x2 - c2) ** 2
        d = (d0 + d2) + d1
        dist = jnp.minimum(dist, d)
        m = jnp.max(dist, axis=1, keepdims=True)
        cand = jnp.where(dist == m, iota, N)
        far = jnp.min(cand, axis=1, keepdims=True)
        return dist, far

    dist0 = jnp.full((B, N), 1e10, dtype=jnp.float32)
    far0 = jnp.zeros((B, 1), dtype=jnp.int32)
    lax.fori_loop(0, G, step, (dist0, far0))


def _fps(xt):
    out = pl.pallas_call(
        _fps_body,
        out_shape=(
            jax.ShapeDtypeStruct((G, B), jnp.int32),
            jax.ShapeDtypeStruct((G, B), jnp.float32),
            jax.ShapeDtypeStruct((G, B), jnp.float32),
            jax.ShapeDtypeStruct((G, B), jnp.float32),
        ),
    )(xt)
    return out


M1 = B * G + B * G * K
M2 = B * G * K
CHUNK = 128
PW1 = M1 // NW
PW2 = M2 // NW
NC1 = PW1 // CHUNK
NC2 = PW2 // CHUNK


def _sc_gather_body(tab128, idx1, out1, idxv, rows, sem):
    wid = lax.axis_index("s") * _SC_INFO.num_cores + lax.axis_index("c")
    base1 = wid * PW1

    def chunk1(t, carry):
        off = base1 + t * CHUNK
        pltpu.sync_copy(idx1.at[pl.ds(off, CHUNK)], idxv)
        pltpu.async_copy(tab128.at[idxv], rows, sem).wait()
        pltpu.sync_copy(rows, out1.at[pl.ds(off, CHUNK)])
        return carry

    lax.fori_loop(0, NC1, chunk1, 0)


@functools.partial(
    pl.kernel,
    out_type=jax.ShapeDtypeStruct((M1, 128), jnp.float32),
    mesh=plsc.VectorSubcoreMesh(core_axis_name="c", subcore_axis_name="s"),
    scratch_types=[
        pltpu.VMEM((CHUNK,), jnp.int32),
        pltpu.VMEM((CHUNK, 128), jnp.float32),
        pltpu.SemaphoreType.DMA,
    ],
)
def _sc_gather(tab128, idx1, out1, idxv, rows, sem):
    _sc_gather_body(tab128, idx1, out1, idxv, rows, sem)


def kernel(xyz, points):
    xt = jnp.transpose(xyz, (2, 0, 1))
    fps_flat, cx, cy, cz = _fps(xt)
    fps_idx = (jnp.transpose(fps_flat) - jnp.arange(B)[:, None] * N).astype(jnp.int32)
    new_xyz = jnp.stack([cx.T, cy.T, cz.T], axis=-1)
    d = -2 * jnp.matmul(new_xyz, jnp.transpose(xyz, (0, 2, 1)))
    d = d + jnp.sum(new_xyz ** 2, -1).reshape(B, G, 1)
    d = d + jnp.sum(xyz ** 2, -1).reshape(B, 1, N)
    _, idx = jax.lax.top_k(-d, K)

    fps_b_major = jnp.transpose(fps_flat).reshape(B * G)
    knn_flat = (idx + jnp.arange(B)[:, None, None] * N).reshape(B * G * K)
    all128_idx = jnp.concatenate([fps_b_major, knn_flat])
    points_flat = points.reshape(B * N, 128)
    out1 = _sc_gather(points_flat, all128_idx)
    new_points = out1[:B * G].reshape(B, G, 128)
    grouped_points = out1[B * G:].reshape(B, G, K, 128)
    bidx3 = jnp.arange(B)[:, None, None]
    grouped_xyz = xyz[bidx3, idx, :]
    return (new_xyz, new_points, grouped_xyz, grouped_points)
